# baseline (speedup 1.0000x reference)
import functools

import jax
import jax.numpy as jnp
import numpy as np
from jax.experimental import pallas as pl
from jax.experimental.pallas import tpu as pltpu

RESO = 128
SCALE = 1.15
CLAMP = 1.1
R2 = RESO * RESO


def _rb_body(x_ref, w0_ref, b0_ref, w1_ref, b1_ref, ws_ref, o_ref):
    x = x_ref[...]
    h = jnp.maximum(x, 0.0) @ w0_ref[...] + b0_ref[...]
    dx = jnp.maximum(h, 0.0) @ w1_ref[...] + b1_ref[...]
    o_ref[...] = x @ ws_ref[...] + dx


@functools.partial(jax.jit, static_argnames=("bm",))
def _resblock_tc(x, W0, b0, W1, b1, Ws, bm=1024):
    M, K = x.shape
    H = W0.shape[1]
    F = W1.shape[1]
    b0r = b0.reshape(1, H)
    b1r = b1.reshape(1, F)
    grid = (M // bm,)
    return pl.pallas_call(
        _rb_body,
        grid=grid,
        in_specs=[
            pl.BlockSpec((bm, K), lambda i: (i, 0)),
            pl.BlockSpec((K, H), lambda i: (0, 0)),
            pl.BlockSpec((1, H), lambda i: (0, 0)),
            pl.BlockSpec((H, F), lambda i: (0, 0)),
            pl.BlockSpec((1, F), lambda i: (0, 0)),
            pl.BlockSpec((K, F), lambda i: (0, 0)),
        ],
        out_specs=pl.BlockSpec((bm, F), lambda i: (i, 0)),
        out_shape=jax.ShapeDtypeStruct((M, F), jnp.float32),
    )(x, W0, b0r, W1, b1r, Ws)


def _plane_indices(p):
    x = jnp.clip(p, -CLAMP, CLAMP) / SCALE / 2.0 + 0.5
    xi = (x * RESO).astype(jnp.int32)
    ix, iy, iz = xi[..., 0], xi[..., 1], xi[..., 2]
    idx_xy = ix + RESO * iy
    idx_yz = iy + RESO * iz
    idx_zx = iz + RESO * ix
    return jnp.stack([idx_xy, idx_yz, idx_zx])


def _scatter_max_jax(c, idx):
    B, N, F = c.shape
    off = (idx + jnp.arange(B, dtype=idx.dtype)[:, None] * R2).reshape(-1)
    seg = jax.ops.segment_max(c.reshape(B * N, F), off, num_segments=B * R2)
    seg = jnp.where(jnp.isneginf(seg), 0.0, seg)
    return seg.reshape(B, R2, F)


def kernel(p, params):
    B, N, _ = p.shape
    M = B * N
    idxs = _plane_indices(p)

    x0 = p.reshape(M, 3)
    net = _resblock_tc(x0, *params[0])

    for prm in params[1:]:
        netB = net.reshape(B, N, -1)
        pooled = 0.0
        for k in range(3):
            scat = _scatter_max_jax(netB, idxs[k])
            gath = jax.vmap(lambda s, i: s[i])(scat, idxs[k])
            pooled = pooled + gath
        x = jnp.concatenate([netB, pooled], axis=-1).reshape(M, -1)
        net = _resblock_tc(x, *prm)

    netB = net.reshape(B, N, -1)
    feas = []
    for k in range(3):
        scat = _scatter_max_jax(netB, idxs[k])
        feas.append(jnp.transpose(scat, (0, 2, 1)).reshape(B, -1, RESO, RESO))
    return tuple(feas)
